# Initial kernel scaffold; baseline (speedup 1.0000x reference)
import jax
import jax.numpy as jnp
from jax.experimental import pallas as pl


def kernel(seq, position_embed):
    raise NotImplementedError("write your pallas kernel here")


# speedup vs baseline: 2.1868x; 2.1868x over previous
import functools

import jax
import jax.numpy as jnp
from jax import lax
from jax.experimental import pallas as pl
from jax.experimental.pallas import tpu as pltpu
from jax.experimental.pallas import tpu_sc as plsc

SEQ_LEN = 8192
EMB_DIM = 1024
BATCH = 4

NUM_CORES = 2
NUM_SUBCORES = 16
NW = NUM_CORES * NUM_SUBCORES
B_TOTAL = BATCH * SEQ_LEN
B_PER_W = B_TOTAL // NW
CHUNK = 64
N_CHUNKS = B_PER_W // CHUNK


def _gather_body(seq_hbm, table_hbm, out_hbm, idx_v, rows_v, sem):
    wid = lax.axis_index("s") * NUM_CORES + lax.axis_index("c")
    base = wid * B_PER_W

    pltpu.sync_copy(seq_hbm.at[wid], idx_v)

    def chunk_step(j, carry):
        pltpu.async_copy(table_hbm.at[idx_v.at[j]], rows_v, sem).wait()
        pltpu.sync_copy(rows_v, out_hbm.at[pl.ds(base + j * CHUNK, CHUNK)])
        return carry

    lax.fori_loop(0, N_CHUNKS, chunk_step, 0)


@jax.jit
def _positional_encoding(seq_grouped, position_embed):
    mesh = plsc.VectorSubcoreMesh(core_axis_name="c", subcore_axis_name="s")
    run = pl.kernel(
        _gather_body,
        out_type=jax.ShapeDtypeStruct((B_TOTAL, EMB_DIM), jnp.float32),
        mesh=mesh,
        scratch_types=[
            pltpu.VMEM((N_CHUNKS, CHUNK), jnp.int32),
            pltpu.VMEM((CHUNK, EMB_DIM), jnp.float32),
            pltpu.SemaphoreType.DMA,
        ],
    )
    return run(seq_grouped, position_embed)


def kernel(seq, position_embed):
    seq_grouped = seq.reshape(NW, N_CHUNKS, CHUNK).astype(jnp.int32)
    out = _positional_encoding(seq_grouped, position_embed)
    return out.reshape(BATCH, SEQ_LEN, EMB_DIM)


# speedup vs baseline: 2.2875x; 1.0460x over previous
import functools

import jax
import jax.numpy as jnp
from jax import lax
from jax.experimental import pallas as pl
from jax.experimental.pallas import tpu as pltpu
from jax.experimental.pallas import tpu_sc as plsc

SEQ_LEN = 8192
EMB_DIM = 1024
BATCH = 4

NUM_CORES = 2
NUM_SUBCORES = 16
NW = NUM_CORES * NUM_SUBCORES
B_TOTAL = BATCH * SEQ_LEN
B_PER_W = B_TOTAL // NW
CHUNK = 32
N_CHUNKS = B_PER_W // CHUNK
N_PAIRS = N_CHUNKS // 2


def _gather_body(seq_hbm, table_hbm, out_hbm, idx_v, rows_v, sem_g, sem_w):
    wid = lax.axis_index("s") * NUM_CORES + lax.axis_index("c")
    base = wid * B_PER_W

    pltpu.sync_copy(seq_hbm.at[wid], idx_v)

    pltpu.async_copy(table_hbm.at[idx_v.at[0]], rows_v.at[0], sem_g)

    def pair_step(g, carry):
        for b in range(2):
            j = 2 * g + b
            out_slice = out_hbm.at[pl.ds(base + j * CHUNK, CHUNK)]
            pltpu.make_async_copy(
                table_hbm.at[idx_v.at[j]], rows_v.at[b], sem_g).wait()
            if b == 1:
                pltpu.make_async_copy(
                    rows_v.at[0],
                    out_hbm.at[pl.ds(base + (j - 1) * CHUNK, CHUNK)],
                    sem_w).wait()
            else:
                @pl.when(g >= 1)
                def _():
                    pltpu.make_async_copy(
                        rows_v.at[1],
                        out_hbm.at[pl.ds(base + (j - 1) * CHUNK, CHUNK)],
                        sem_w).wait()
            if b == 0:
                pltpu.async_copy(
                    table_hbm.at[idx_v.at[j + 1]], rows_v.at[1], sem_g)
            else:
                @pl.when(g < N_PAIRS - 1)
                def _():
                    pltpu.async_copy(
                        table_hbm.at[idx_v.at[j + 1]], rows_v.at[0], sem_g)
            pltpu.async_copy(rows_v.at[b], out_slice, sem_w)
        return carry

    lax.fori_loop(0, N_PAIRS, pair_step, 0)

    last = N_CHUNKS - 1
    pltpu.make_async_copy(
        rows_v.at[1], out_hbm.at[pl.ds(base + last * CHUNK, CHUNK)],
        sem_w).wait()


@jax.jit
def _positional_encoding(seq_grouped, position_embed):
    mesh = plsc.VectorSubcoreMesh(core_axis_name="c", subcore_axis_name="s")
    run = pl.kernel(
        _gather_body,
        out_type=jax.ShapeDtypeStruct((B_TOTAL, EMB_DIM), jnp.float32),
        mesh=mesh,
        scratch_types=[
            pltpu.VMEM((N_CHUNKS, CHUNK), jnp.int32),
            pltpu.VMEM((2, CHUNK, EMB_DIM), jnp.float32),
            pltpu.SemaphoreType.DMA,
            pltpu.SemaphoreType.DMA,
        ],
    )
    return run(seq_grouped, position_embed)


def kernel(seq, position_embed):
    seq_grouped = seq.reshape(NW, N_CHUNKS, CHUNK).astype(jnp.int32)
    out = _positional_encoding(seq_grouped, position_embed)
    return out.reshape(BATCH, SEQ_LEN, EMB_DIM)
